# baseline (speedup 1.0000x reference)
import functools

import jax
import jax.numpy as jnp
from jax import lax
from jax.experimental import pallas as pl
from jax.experimental.pallas import tpu as pltpu
from jax.experimental.pallas import tpu_sc as plsc

_PACK = 4
_PIECE_I = 4
_NBUF = 2


def kernel(context_mapping, table):
    n, l = context_mapping.shape
    d = table.shape[1]
    lp = l // _PACK
    bp = n * lp

    info = plsc.get_sparse_core_info()
    nw = info.num_cores * info.num_subcores
    lanes = info.num_lanes

    rows_pw = n // nw
    piece = _PIECE_I * lp
    n_pieces = rows_pw // _PIECE_I
    n_outer = n_pieces // _NBUF
    pad = (-piece) % lanes
    n_mul = (piece + pad) // lanes
    spans = []
    s = 0
    while s < piece:
        ln = min(128, piece - s)
        spans.append((s, ln))
        s += ln

    t_zero = jnp.broadcast_to(table[0:1, :], (lp, d))
    t_k = [table[k:l:_PACK, :] for k in range(_PACK)]
    combos = []
    for c in range(2 ** _PACK):
        parts = [t_k[k] if (c >> k) & 1 else t_zero for k in range(_PACK)]
        combos.append(jnp.concatenate(parts, axis=1))
    ptable = jnp.concatenate(combos, axis=0)

    weights = jnp.asarray([1 << k for k in range(_PACK)], jnp.int32)
    code = jnp.einsum(
        "igk,k->ig",
        context_mapping.reshape(n, lp, _PACK).astype(jnp.int32),
        weights).reshape(bp)
    jseq = jnp.concatenate([
        jnp.tile(jnp.arange(lp, dtype=jnp.int32), _PIECE_I),
        jnp.zeros((pad,), jnp.int32)])

    scratch = [pltpu.VMEM((piece + pad,), jnp.int32)]
    for _ in range(_NBUF):
        scratch += [
            pltpu.VMEM((piece + pad,), jnp.int32),
            pltpu.VMEM((piece + pad,), jnp.int32),
            pltpu.VMEM((piece, _PACK * d), jnp.float32),
            pltpu.SemaphoreType.DMA,
            pltpu.SemaphoreType.DMA,
            pltpu.SemaphoreType.DMA]

    @functools.partial(
        pl.kernel,
        mesh=plsc.VectorSubcoreMesh(core_axis_name="c", subcore_axis_name="s"),
        out_type=jax.ShapeDtypeStruct((bp, _PACK * d), jnp.float32),
        scratch_types=scratch,
    )
    def sc_gather(code_hbm, jseq_hbm, ptable_hbm, out_hbm, jseq_v, *bufs):
        wid = lax.axis_index("s") * info.num_cores + lax.axis_index("c")
        base = wid * rows_pw * lp
        rings = [bufs[6 * i:6 * i + 6] for i in range(_NBUF)]
        pltpu.sync_copy(jseq_hbm, jseq_v)

        def fire_m(t, r):
            off = pl.multiple_of(base + t * piece, 8)
            pltpu.async_copy(
                code_hbm.at[pl.ds(off, piece)], r[0].at[pl.ds(0, piece)],
                r[3])

        def drain_m(r):
            pltpu.make_async_copy(
                code_hbm.at[pl.ds(0, piece)], r[0].at[pl.ds(0, piece)],
                r[3]).wait()

        def fire_g(r):
            for (s0, ln) in spans:
                pltpu.async_copy(
                    ptable_hbm.at[r[1].at[pl.ds(s0, ln)]],
                    r[2].at[pl.ds(s0, ln)], r[4])

        def drain_g(r):
            for (s0, ln) in spans:
                pltpu.make_async_copy(
                    ptable_hbm.at[r[1].at[pl.ds(s0, ln)]],
                    r[2].at[pl.ds(s0, ln)], r[4]).wait()

        def fire_o(t, r):
            off = pl.multiple_of(base + t * piece, 8)
            pltpu.async_copy(r[2], out_hbm.at[pl.ds(off, piece)], r[5])

        def drain_o(r):
            pltpu.make_async_copy(
                r[2], out_hbm.at[pl.ds(base, piece)], r[5]).wait()

        def compute_idx(r):
            for v in range(n_mul):
                sl = pl.ds(v * lanes, lanes)
                r[1][sl] = r[0][sl] * lp + jseq_v[sl]

        for bi in range(_NBUF):
            fire_m(bi, rings[bi])

        def body(g, carry):
            for bi in range(_NBUF):
                t = _NBUF * g + bi
                r = rings[bi]
                rprev = rings[(bi - 1) % _NBUF]
                drain_m(r)

                @pl.when(g >= 1)
                def _():
                    drain_o(r)

                compute_idx(r)
                fire_g(r)

                if bi >= 1:
                    drain_g(rprev)
                    fire_o(t - 1, rprev)
                else:
                    @pl.when(g >= 1)
                    def _():
                        drain_g(rprev)
                        fire_o(t - 1, rprev)

                @pl.when(g <= n_outer - 2)
                def _():
                    fire_m(t + _NBUF, r)
            return carry

        lax.fori_loop(0, n_outer, body, 0)

        last = rings[(n_pieces - 1) % _NBUF]
        drain_g(last)
        fire_o(n_pieces - 1, last)
        for bi in range(_NBUF):
            drain_o(rings[bi])

    out = sc_gather(code, jseq, ptable)
    return out.reshape(n, l, d)


# speedup vs baseline: 1.0649x; 1.0649x over previous
import functools

import jax
import jax.numpy as jnp
from jax import lax
from jax.experimental import pallas as pl
from jax.experimental.pallas import tpu as pltpu
from jax.experimental.pallas import tpu_sc as plsc

_PIECE_I = 4
_GSUB = 80


def kernel(context_mapping, table):
    n, l = context_mapping.shape
    d = table.shape[1]
    l2 = l // 2
    b2 = n * l2

    info = plsc.get_sparse_core_info()
    nw = info.num_cores * info.num_subcores
    lanes = info.num_lanes

    rows_pw = n // nw
    piece = _PIECE_I * l2
    n_pieces = rows_pw // _PIECE_I
    n_mul = piece // lanes
    n_gsub = piece // _GSUB

    t_even = table[0:l:2, :]
    t_odd = table[1:l:2, :]
    t_zero = jnp.broadcast_to(table[0:1, :], (l2, d))
    ptable = jnp.concatenate([
        jnp.concatenate([t_zero, t_zero], axis=1),
        jnp.concatenate([t_even, t_zero], axis=1),
        jnp.concatenate([t_zero, t_odd], axis=1),
        jnp.concatenate([t_even, t_odd], axis=1),
    ], axis=0)

    me_flat = context_mapping[:, 0::2].reshape(b2).astype(jnp.int32)
    mo_flat = context_mapping[:, 1::2].reshape(b2).astype(jnp.int32)
    jseq = jnp.tile(jnp.arange(l2, dtype=jnp.int32), _PIECE_I)

    @functools.partial(
        pl.kernel,
        mesh=plsc.VectorSubcoreMesh(core_axis_name="c", subcore_axis_name="s"),
        out_type=jax.ShapeDtypeStruct((b2, 2 * d), jnp.float32),
        scratch_types=[
            pltpu.VMEM((piece,), jnp.int32),
            pltpu.VMEM((piece,), jnp.int32),
            pltpu.VMEM((piece,), jnp.int32),
            pltpu.VMEM((piece,), jnp.int32),
            pltpu.VMEM((piece, 2 * d), jnp.float32),
            pltpu.SemaphoreType.DMA,
        ],
    )
    def sc_gather(me_hbm, mo_hbm, jseq_hbm, ptable_hbm, out_hbm,
                  jseq_v, me_v, mo_v, idx_v, rows_v, sem):
        wid = lax.axis_index("s") * info.num_cores + lax.axis_index("c")
        base = wid * rows_pw * l2
        pltpu.sync_copy(jseq_hbm, jseq_v)

        def body(t, carry):
            off = pl.multiple_of(base + t * piece, piece)
            pltpu.sync_copy(me_hbm.at[pl.ds(off, piece)], me_v)
            pltpu.sync_copy(mo_hbm.at[pl.ds(off, piece)], mo_v)
            for v in range(n_mul):
                sl = pl.ds(v * lanes, lanes)
                idx_v[sl] = (me_v[sl] + 2 * mo_v[sl]) * l2 + jseq_v[sl]
            copies = []
            for g in range(n_gsub):
                gs = pl.ds(g * _GSUB, _GSUB)
                copies.append(pltpu.async_copy(
                    ptable_hbm.at[idx_v.at[gs]], rows_v.at[gs], sem))
            for c in copies:
                c.wait()
            pltpu.sync_copy(rows_v, out_hbm.at[pl.ds(off, piece)])
            return carry

        lax.fori_loop(0, n_pieces, body, 0)

    out = sc_gather(me_flat, mo_flat, jseq, ptable)
    return out.reshape(n, l, d)
